# Initial kernel scaffold; baseline (speedup 1.0000x reference)
import jax
import jax.numpy as jnp
from jax.experimental import pallas as pl


def kernel(image1, image2):
    raise NotImplementedError("write your pallas kernel here")


# speedup vs baseline: 4.6783x; 4.6783x over previous
import jax
import jax.numpy as jnp
from jax import lax
from jax.experimental import pallas as pl
from jax.experimental.pallas import tpu as pltpu

_B, _C, _H, _W = 8, 8, 1024, 1024
_N = _C * _H * _W
_ND = _H * _W
_ROWS = _B * _C
_RPS = 2
_STEPS = _ROWS // _RPS


def _moment_kernel(x1_ref, x2_ref, o_ref, acc_ref):
    j = pl.program_id(0)

    @pl.when(j == 0)
    def _():
        acc_ref[...] = jnp.zeros_like(acc_ref)

    sub = lax.broadcasted_iota(jnp.int32, (8, 128), 0)
    lane = lax.broadcasted_iota(jnp.int32, (8, 128), 1)

    contrib = jnp.zeros((8, 128), jnp.float32)
    for i in range(_RPS):
        r = j * _RPS + i
        b = r // _C
        c = r % _C
        on_diag = c == b
        row_mask = sub == b
        for img, ref in ((0, x1_ref), (1, x2_ref)):
            x = ref[i]
            s1 = jnp.sum(x)
            s2 = jnp.sum(x * x)
            for q, val in ((0, s1), (1, s2)):
                lq = img * 2 + q
                contrib = contrib + jnp.where(
                    row_mask & (lane == lq), val, 0.0)
                contrib = contrib + jnp.where(
                    row_mask & (lane == lq + 4) & on_diag, val, 0.0)
    acc_ref[...] += contrib

    @pl.when(j == _STEPS - 1)
    def _():
        acc = acc_ref[...]

        def img_moment(base):
            s1 = acc[:, base + 0:base + 1]
            s2 = acc[:, base + 1:base + 2]
            d1 = acc[:, base + 4:base + 5]
            d2 = acc[:, base + 5:base + 6]
            mean = s1 * (1.0 / _N)
            var = (s2 - s1 * mean) * (1.0 / (_N - 1))
            mom = (d2 - 2.0 * mean * d1 + _ND * mean * mean) / var
            return jnp.sum(mom) * (1.0 / (float(_N) ** 2))

        o_ref[0, 0] = jnp.abs(img_moment(0) - img_moment(2))


def kernel(image1, image2, *, interpret=False):
    x1 = image1.reshape(_ROWS, _H, _W)
    x2 = image2.reshape(_ROWS, _H, _W)
    out = pl.pallas_call(
        _moment_kernel,
        grid=(_STEPS,),
        in_specs=[
            pl.BlockSpec((_RPS, _H, _W), lambda j: (j, 0, 0)),
            pl.BlockSpec((_RPS, _H, _W), lambda j: (j, 0, 0)),
        ],
        out_specs=pl.BlockSpec((1, 1), lambda j: (0, 0),
                               memory_space=pltpu.SMEM),
        out_shape=jax.ShapeDtypeStruct((1, 1), jnp.float32),
        scratch_shapes=[pltpu.VMEM((8, 128), jnp.float32)],
        compiler_params=pltpu.CompilerParams(
            dimension_semantics=("arbitrary",),
            vmem_limit_bytes=40 * 1024 * 1024,
        ),
        name="calculate_moment",
        interpret=interpret,
    )(x1, x2)
    return out[0, 0]
